# Initial kernel scaffold; baseline (speedup 1.0000x reference)
import jax
import jax.numpy as jnp
from jax.experimental import pallas as pl


def kernel(pos, node_size_x, node_size_y, initial_density_map):
    raise NotImplementedError("write your pallas kernel here")


# speedup vs baseline: 53.5966x; 53.5966x over previous
import functools

import jax
import jax.numpy as jnp
from jax import lax
from jax.experimental import pallas as pl
from jax.experimental.pallas import tpu as pltpu
from jax.experimental.pallas import tpu_sc as plsc

NBX = 512
NBY = 512
BSX = 1.0 / NBX
BSY = 1.0 / NBY
THX = BSX
THY = BSY
TARGET_DENSITY = 0.8
BIN_AREA = BSX * BSY
DENS_SCALE = 0.25 / (THX * THY)
INV_BIN_AREA = float(NBX * NBY)

NC = 2
NS = 16
NW = NC * NS
CHUNK = 2048
CPW = 16
NPAD = NW * CHUNK * CPW
NROWS = 9 * CHUNK // 128


def _compute_group(px, py, sx, sy):
    cx = px + 0.5 * sx
    cy = py + 0.5 * sy
    dens = (sx * sy) * DENS_SCALE

    tx = (cx - THX) * float(NBX)
    ty = (cy - THY) * float(NBY)
    ixt = tx.astype(jnp.int32)
    iyt = ty.astype(jnp.int32)
    lox = jnp.where(ixt.astype(jnp.float32) > tx, ixt - 1, ixt)
    loy = jnp.where(iyt.astype(jnp.float32) > ty, iyt - 1, iyt)
    lofx = lox.astype(jnp.float32)
    lofy = loy.astype(jnp.float32)

    cx_p = cx + THX
    cx_m = cx - THX
    cy_p = cy + THY
    cy_m = cy - THY

    wx, colx, oy, rowy = [], [], [], []
    for k in range(3):
        bx = lox + k
        bl = (lofx + float(k)) * BSX
        ox = jnp.minimum(cx_p, bl + BSX) - jnp.maximum(cx_m, bl)
        inb = (bx >= 0) & (bx < NBX)
        ox = jnp.where(inb, jnp.maximum(ox, 0.0), 0.0)
        wx.append(ox * dens)
        colx.append(jnp.clip(bx, 0, NBX - 1) * NBY)

        by = loy + k
        bly = (lofy + float(k)) * BSY
        o = jnp.minimum(cy_p, bly + BSY) - jnp.maximum(cy_m, bly)
        inby = (by >= 0) & (by < NBY)
        oy.append(jnp.where(inby, jnp.maximum(o, 0.0), 0.0))
        rowy.append(jnp.clip(by, 0, NBY - 1))
    return wx, colx, oy, rowy


@functools.partial(
    pl.kernel,
    out_type=jax.ShapeDtypeStruct((NC, NBX * NBY), jnp.float32),
    mesh=plsc.VectorSubcoreMesh(core_axis_name="c", subcore_axis_name="s"),
    scratch_types=[
        pltpu.VMEM((CHUNK,), jnp.float32),
        pltpu.VMEM((CHUNK,), jnp.float32),
        pltpu.VMEM((CHUNK,), jnp.float32),
        pltpu.VMEM((CHUNK,), jnp.float32),
        pltpu.VMEM((9 * CHUNK,), jnp.int32),
        pltpu.VMEM((9 * CHUNK,), jnp.float32),
        pltpu.VMEM_SHARED((NBX * NBY,), jnp.float32),
    ],
)
def _sc_scatter(px_hbm, py_hbm, sx_hbm, sy_hbm, zeros_hbm, out_hbm,
                px_v, py_v, sx_v, sy_v, idx_v, val_v, map_sh):
    c = lax.axis_index("c")
    s = lax.axis_index("s")
    wid = c * NS + s

    seg = NBX * NBY // NS
    pltpu.sync_copy(zeros_hbm.at[pl.ds(s * seg, seg)],
                    map_sh.at[pl.ds(s * seg, seg)])
    plsc.subcore_barrier()

    def chunk_body(i, carry):
        base = wid * (CHUNK * CPW) + i * CHUNK
        pltpu.sync_copy(px_hbm.at[pl.ds(base, CHUNK)], px_v)
        pltpu.sync_copy(py_hbm.at[pl.ds(base, CHUNK)], py_v)
        pltpu.sync_copy(sx_hbm.at[pl.ds(base, CHUNK)], sx_v)
        pltpu.sync_copy(sy_hbm.at[pl.ds(base, CHUNK)], sy_v)

        def group_body(j, carry2):
            o16 = j * 16
            px = px_v[pl.ds(o16, 16)]
            py = py_v[pl.ds(o16, 16)]
            sx = sx_v[pl.ds(o16, 16)]
            sy = sy_v[pl.ds(o16, 16)]
            wx, colx, oy, rowy = _compute_group(px, py, sx, sy)
            for kx in range(3):
                for ky in range(3):
                    off = (kx * 3 + ky) * CHUNK + o16
                    idx_v[pl.ds(off, 16)] = colx[kx] + rowy[ky]
                    val_v[pl.ds(off, 16)] = wx[kx] * oy[ky]
            return carry2

        lax.fori_loop(0, CHUNK // 16, group_body, 0, unroll=False)
        pltpu.sync_copy(val_v, map_sh.at[idx_v], add=True)
        return carry

    lax.fori_loop(0, CPW, chunk_body, 0, unroll=False)
    plsc.subcore_barrier()

    @pl.when(s == 0)
    def _():
        pltpu.sync_copy(map_sh, out_hbm.at[c])


def _reduce_body(parts_ref, init_ref, cost_ref, maxd_ref):
    d = parts_ref[0] + parts_ref[1] + init_ref[...]
    cost_ref[...] = jnp.sum(jnp.maximum(d - TARGET_DENSITY * BIN_AREA, 0.0)).reshape(1, 1)
    maxd_ref[...] = (jnp.max(d) * INV_BIN_AREA).reshape(1, 1)


def kernel(pos, node_size_x, node_size_y, initial_density_map):
    n = node_size_x.shape[0]
    pad = NPAD - n
    px = jnp.concatenate([pos[:n], jnp.full((pad,), 0.5, jnp.float32)])
    py = jnp.concatenate([pos[n:], jnp.full((pad,), 0.5, jnp.float32)])
    sx = jnp.concatenate([node_size_x, jnp.zeros((pad,), jnp.float32)])
    sy = jnp.concatenate([node_size_y, jnp.zeros((pad,), jnp.float32)])
    zeros = jnp.zeros((NBX * NBY,), jnp.float32)

    parts = _sc_scatter(px, py, sx, sy, zeros)

    cost, maxd = pl.pallas_call(
        _reduce_body,
        out_shape=(
            jax.ShapeDtypeStruct((1, 1), jnp.float32),
            jax.ShapeDtypeStruct((1, 1), jnp.float32),
        ),
    )(parts.reshape(NC, NBX, NBY), initial_density_map)
    return (cost.reshape(1), maxd.reshape(1))


# speedup vs baseline: 78.4538x; 1.4638x over previous
import functools

import jax
import jax.numpy as jnp
from jax import lax
from jax.experimental import pallas as pl
from jax.experimental.pallas import tpu as pltpu
from jax.experimental.pallas import tpu_sc as plsc

NBX = 512
NBY = 512
BSX = 1.0 / NBX
BSY = 1.0 / NBY
THX = BSX
THY = BSY
TARGET_DENSITY = 0.8
BIN_AREA = BSX * BSY
DENS_SCALE = 0.25 / (THX * THY)
INV_BIN_AREA = float(NBX * NBY)

NC = 2
NS = 16
NW = NC * NS
CHUNK = 2048
CPW = 16
NPAD = NW * CHUNK * CPW
NROWS = 9 * CHUNK // 128


def _compute_group(px, py, sx, sy):
    cx = px + 0.5 * sx
    cy = py + 0.5 * sy
    dens = (sx * sy) * DENS_SCALE

    tx = (cx - THX) * float(NBX)
    ty = (cy - THY) * float(NBY)
    ixt = tx.astype(jnp.int32)
    iyt = ty.astype(jnp.int32)
    lox = jnp.where(ixt.astype(jnp.float32) > tx, ixt - 1, ixt)
    loy = jnp.where(iyt.astype(jnp.float32) > ty, iyt - 1, iyt)
    lofx = lox.astype(jnp.float32)
    lofy = loy.astype(jnp.float32)

    cx_p = cx + THX
    cx_m = cx - THX
    cy_p = cy + THY
    cy_m = cy - THY

    wx, colx, oy, rowy = [], [], [], []
    for k in range(3):
        bx = lox + k
        bl = (lofx + float(k)) * BSX
        ox = jnp.minimum(cx_p, bl + BSX) - jnp.maximum(cx_m, bl)
        inb = (bx >= 0) & (bx < NBX)
        ox = jnp.where(inb, jnp.maximum(ox, 0.0), 0.0)
        wx.append(ox * dens)
        colx.append(jnp.clip(bx, 0, NBX - 1) * NBY)

        by = loy + k
        bly = (lofy + float(k)) * BSY
        o = jnp.minimum(cy_p, bly + BSY) - jnp.maximum(cy_m, bly)
        inby = (by >= 0) & (by < NBY)
        oy.append(jnp.where(inby, jnp.maximum(o, 0.0), 0.0))
        rowy.append(jnp.clip(by, 0, NBY - 1))
    return wx, colx, oy, rowy


@functools.partial(
    pl.kernel,
    out_type=jax.ShapeDtypeStruct((NC, NBX * NBY), jnp.float32),
    mesh=plsc.VectorSubcoreMesh(core_axis_name="c", subcore_axis_name="s"),
    scratch_types=[
        [pltpu.VMEM((CHUNK,), jnp.float32) for _ in range(2)],
        [pltpu.VMEM((CHUNK,), jnp.float32) for _ in range(2)],
        [pltpu.VMEM((CHUNK,), jnp.float32) for _ in range(2)],
        [pltpu.VMEM((CHUNK,), jnp.float32) for _ in range(2)],
        [pltpu.VMEM((9 * CHUNK,), jnp.int32) for _ in range(2)],
        [pltpu.VMEM((9 * CHUNK,), jnp.float32) for _ in range(2)],
        pltpu.VMEM_SHARED((NBX * NBY,), jnp.float32),
        [pltpu.SemaphoreType.DMA for _ in range(2)],
        [pltpu.SemaphoreType.DMA for _ in range(2)],
    ],
)
def _sc_scatter(px_hbm, py_hbm, sx_hbm, sy_hbm, zeros_hbm, out_hbm,
                px_v, py_v, sx_v, sy_v, idx_v, val_v, map_sh,
                sem_in, sem_sc):
    c = lax.axis_index("c")
    s = lax.axis_index("s")
    wid = c * NS + s
    wbase = wid * (CHUNK * CPW)

    seg = NBX * NBY // NS
    pltpu.sync_copy(zeros_hbm.at[pl.ds(s * seg, seg)],
                    map_sh.at[pl.ds(s * seg, seg)])
    plsc.subcore_barrier()

    def start_loads(chunk_idx, b):
        base = wbase + chunk_idx * CHUNK
        pltpu.async_copy(px_hbm.at[pl.ds(base, CHUNK)], px_v[b], sem_in[b])
        pltpu.async_copy(py_hbm.at[pl.ds(base, CHUNK)], py_v[b], sem_in[b])
        pltpu.async_copy(sx_hbm.at[pl.ds(base, CHUNK)], sx_v[b], sem_in[b])
        pltpu.async_copy(sy_hbm.at[pl.ds(base, CHUNK)], sy_v[b], sem_in[b])

    def wait_loads(b):
        for dst in (px_v[b], py_v[b], sx_v[b], sy_v[b]):
            pltpu.make_async_copy(px_hbm.at[pl.ds(0, CHUNK)], dst,
                                  sem_in[b]).wait()

    def wait_scatter(b):
        pltpu.make_async_copy(val_v[b], map_sh.at[idx_v[b]],
                              sem_sc[b]).wait()

    def compute_chunk(b):
        def group_body(j, carry2):
            o16 = j * 16
            px = px_v[b][pl.ds(o16, 16)]
            py = py_v[b][pl.ds(o16, 16)]
            sx = sx_v[b][pl.ds(o16, 16)]
            sy = sy_v[b][pl.ds(o16, 16)]
            wx, colx, oy, rowy = _compute_group(px, py, sx, sy)
            for kx in range(3):
                for ky in range(3):
                    off = (kx * 3 + ky) * CHUNK + o16
                    idx_v[b][pl.ds(off, 16)] = colx[kx] + rowy[ky]
                    val_v[b][pl.ds(off, 16)] = wx[kx] * oy[ky]
            return carry2

        lax.fori_loop(0, CHUNK // 16, group_body, 0, unroll=False)

    start_loads(0, 0)

    def pair_body(ip, carry):
        for b in range(2):
            i = ip * 2 + b
            wait_loads(b)

            @pl.when(i + 1 < CPW)
            def _():
                start_loads(i + 1, 1 - b)

            @pl.when(i >= 2)
            def _():
                wait_scatter(b)

            compute_chunk(b)
            pltpu.async_copy(val_v[b], map_sh.at[idx_v[b]], sem_sc[b],
                             add=True)
        return carry

    lax.fori_loop(0, CPW // 2, pair_body, 0, unroll=False)
    wait_scatter(0)
    wait_scatter(1)
    plsc.subcore_barrier()

    @pl.when(s == 0)
    def _():
        pltpu.sync_copy(map_sh, out_hbm.at[c])


def _reduce_body(parts_ref, init_ref, cost_ref, maxd_ref):
    d = parts_ref[0] + parts_ref[1] + init_ref[...]
    cost_ref[...] = jnp.sum(jnp.maximum(d - TARGET_DENSITY * BIN_AREA, 0.0)).reshape(1, 1)
    maxd_ref[...] = (jnp.max(d) * INV_BIN_AREA).reshape(1, 1)


def kernel(pos, node_size_x, node_size_y, initial_density_map):
    n = node_size_x.shape[0]
    pad = NPAD - n
    px = jnp.concatenate([pos[:n], jnp.full((pad,), 0.5, jnp.float32)])
    py = jnp.concatenate([pos[n:], jnp.full((pad,), 0.5, jnp.float32)])
    sx = jnp.concatenate([node_size_x, jnp.zeros((pad,), jnp.float32)])
    sy = jnp.concatenate([node_size_y, jnp.zeros((pad,), jnp.float32)])
    zeros = jnp.zeros((NBX * NBY,), jnp.float32)

    parts = _sc_scatter(px, py, sx, sy, zeros)

    cost, maxd = pl.pallas_call(
        _reduce_body,
        out_shape=(
            jax.ShapeDtypeStruct((1, 1), jnp.float32),
            jax.ShapeDtypeStruct((1, 1), jnp.float32),
        ),
    )(parts.reshape(NC, NBX, NBY), initial_density_map)
    return (cost.reshape(1), maxd.reshape(1))
